# Initial kernel scaffold; baseline (speedup 1.0000x reference)
import functools

import jax
import jax.numpy as jnp
from jax import lax
from jax.experimental import pallas as pl
from jax.experimental.pallas import tpu as pltpu
from jax.experimental.pallas import tpu_sc as plsc

N_NODES = 10000
N_EDGES = 320000
D_IN = 128
D_HID = 64
D_OUT = 40
D_OUT_PAD = 48

NC = 2
NS = 16
NW = NC * NS

R = 10240
ROWS_PER_TILE = R // NS
PAD_ROW = N_NODES
CH = 128
TCH = 80
EPAD = NW * TCH * CH

BLK = 256
GRID = R // BLK

_MESH = plsc.VectorSubcoreMesh(
    core_axis_name="c", subcore_axis_name="s", num_cores=NC, num_subcores=NS
)


def _zero_acc(zeros_hbm, acc, s):
    pltpu.sync_copy(zeros_hbm, acc.at[pl.ds(s * ROWS_PER_TILE, ROWS_PER_TILE)])


def _writeback(acc, out_hbm, c, s):
    base = s * ROWS_PER_TILE
    pltpu.sync_copy(
        acc.at[pl.ds(base, ROWS_PER_TILE)],
        out_hbm.at[c, pl.ds(base, ROWS_PER_TILE)],
    )


def _make_deg_kernel():
    @functools.partial(
        pl.kernel,
        mesh=_MESH,
        out_type=jax.ShapeDtypeStruct((NC, R, 16), jnp.float32),
        scratch_types=[
            pltpu.VMEM((TCH, CH), jnp.int32),
            pltpu.VMEM((CH, 16), jnp.float32),
            pltpu.VMEM_SHARED((R, 16), jnp.float32),
        ],
    )
    def deg_kernel(dst_hbm, ones_hbm, zeros_hbm, out_hbm, dst_v, ones_v, acc):
        c = lax.axis_index("c")
        s = lax.axis_index("s")
        w = c * NS + s
        _zero_acc(zeros_hbm, acc, s)
        pltpu.sync_copy(ones_hbm, ones_v)
        pltpu.sync_copy(dst_hbm.at[w], dst_v)
        plsc.subcore_barrier()

        def body(j, carry):
            pltpu.sync_copy(ones_v, acc.at[dst_v.at[j]], add=True)
            return carry

        lax.fori_loop(0, TCH, body, 0)
        plsc.subcore_barrier()
        _writeback(acc, out_hbm, c, s)

    return deg_kernel


def _make_agg_kernel(d):
    @functools.partial(
        pl.kernel,
        mesh=_MESH,
        out_type=jax.ShapeDtypeStruct((NC, R, d), jnp.float32),
        scratch_types=[
            pltpu.VMEM((TCH, CH), jnp.int32),
            pltpu.VMEM((TCH, CH), jnp.int32),
            pltpu.VMEM((CH, d), jnp.float32),
            pltpu.VMEM((CH, d), jnp.float32),
            pltpu.VMEM_SHARED((R, d), jnp.float32),
            pltpu.SemaphoreType.DMA,
            pltpu.SemaphoreType.DMA,
        ],
    )
    def agg_kernel(
        table_hbm, src_hbm, dst_hbm, zeros_hbm, out_hbm,
        src_v, dst_v, rows0, rows1, acc, sem0, sem1,
    ):
        c = lax.axis_index("c")
        s = lax.axis_index("s")
        w = c * NS + s
        _zero_acc(zeros_hbm, acc, s)
        pltpu.sync_copy(src_hbm.at[w], src_v)
        pltpu.sync_copy(dst_hbm.at[w], dst_v)
        plsc.subcore_barrier()

        bufs = (rows0, rows1)
        sems = (sem0, sem1)
        pltpu.async_copy(table_hbm.at[src_v.at[0]], rows0, sem0)
        pltpu.async_copy(table_hbm.at[src_v.at[1]], rows1, sem1)

        def body(j2, carry):
            for t in range(2):
                j = 2 * j2 + t
                pltpu.make_async_copy(
                    table_hbm.at[src_v.at[j]], bufs[t], sems[t]
                ).wait()
                pltpu.sync_copy(bufs[t], acc.at[dst_v.at[j]], add=True)

                @pl.when(j2 < TCH // 2 - 1)
                def _():
                    pltpu.async_copy(
                        table_hbm.at[src_v.at[j + 2]], bufs[t], sems[t]
                    )

            return carry

        lax.fori_loop(0, TCH // 2, body, 0)
        plsc.subcore_barrier()
        _writeback(acc, out_hbm, c, s)

    return agg_kernel


_deg_call = _make_deg_kernel()
_agg64_call = _make_agg_kernel(D_HID)
_agg48_call = _make_agg_kernel(D_OUT_PAD)


def _mm1_body(xb, w1b, degb, h1s_out, dis_out):
    deg = degb[0, :, 0:1] + degb[1, :, 0:1] + 1.0
    dis = lax.rsqrt(deg)
    h = jnp.dot(xb[...], w1b[...], preferred_element_type=jnp.float32)
    h1s_out[...] = h * dis
    dis_out[...] = jnp.broadcast_to(dis, (BLK, 16))


def _mm2_body(aggb, h1sb, disb, b1b, w2b, h2s_out):
    i = pl.program_id(0)
    dis = disb[:, 0:1]
    agg = aggb[0] + aggb[1] + h1sb[...]
    z = jnp.maximum(agg * dis + b1b[...], 0.0)
    h2 = jnp.dot(z, w2b[...], preferred_element_type=jnp.float32) * dis
    rows = lax.broadcasted_iota(jnp.int32, (BLK, D_OUT_PAD), 0) + i * BLK
    h2s_out[...] = jnp.where(rows < N_NODES, h2, 0.0)


def _post_body(aggb, h2sb, disb, b2b, out):
    agg = aggb[0] + aggb[1] + h2sb[...]
    out[...] = agg * disb[:, 0:1] + b2b[...]


def _mm1(x_pad, W1, degp):
    return pl.pallas_call(
        _mm1_body,
        grid=(GRID,),
        in_specs=[
            pl.BlockSpec((BLK, D_IN), lambda i: (i, 0)),
            pl.BlockSpec((D_IN, D_HID), lambda i: (0, 0)),
            pl.BlockSpec((NC, BLK, 16), lambda i: (0, i, 0)),
        ],
        out_specs=[
            pl.BlockSpec((BLK, D_HID), lambda i: (i, 0)),
            pl.BlockSpec((BLK, 16), lambda i: (i, 0)),
        ],
        out_shape=[
            jax.ShapeDtypeStruct((R, D_HID), jnp.float32),
            jax.ShapeDtypeStruct((R, 16), jnp.float32),
        ],
    )(x_pad, W1, degp)


def _mm2(aggp, h1s, dis16, b1r, W2p):
    return pl.pallas_call(
        _mm2_body,
        grid=(GRID,),
        in_specs=[
            pl.BlockSpec((NC, BLK, D_HID), lambda i: (0, i, 0)),
            pl.BlockSpec((BLK, D_HID), lambda i: (i, 0)),
            pl.BlockSpec((BLK, 16), lambda i: (i, 0)),
            pl.BlockSpec((1, D_HID), lambda i: (0, 0)),
            pl.BlockSpec((D_HID, D_OUT_PAD), lambda i: (0, 0)),
        ],
        out_specs=pl.BlockSpec((BLK, D_OUT_PAD), lambda i: (i, 0)),
        out_shape=jax.ShapeDtypeStruct((R, D_OUT_PAD), jnp.float32),
    )(aggp, h1s, dis16, b1r, W2p)


def _post(aggp, h2s, dis16, b2r):
    return pl.pallas_call(
        _post_body,
        grid=(GRID,),
        in_specs=[
            pl.BlockSpec((NC, BLK, D_OUT_PAD), lambda i: (0, i, 0)),
            pl.BlockSpec((BLK, D_OUT_PAD), lambda i: (i, 0)),
            pl.BlockSpec((BLK, 16), lambda i: (i, 0)),
            pl.BlockSpec((1, D_OUT_PAD), lambda i: (0, 0)),
        ],
        out_specs=pl.BlockSpec((BLK, D_OUT_PAD), lambda i: (i, 0)),
        out_shape=jax.ShapeDtypeStruct((R, D_OUT_PAD), jnp.float32),
    )(aggp, h2s, dis16, b2r)


@jax.jit
def _run(x, edge_index, W1, b1, W2, b2):
    edge_index = edge_index.astype(jnp.int32)
    pad = jnp.full((EPAD - N_EDGES,), PAD_ROW, dtype=jnp.int32)
    src3 = jnp.concatenate([edge_index[0], pad]).reshape(NW, TCH, CH)
    dst3 = jnp.concatenate([edge_index[1], pad]).reshape(NW, TCH, CH)

    x_pad = jnp.pad(x, ((0, R - N_NODES), (0, 0)))
    W2p = jnp.pad(W2, ((0, 0), (0, D_OUT_PAD - D_OUT)))
    b1r = b1.reshape(1, D_HID)
    b2r = jnp.pad(b2, (0, D_OUT_PAD - D_OUT)).reshape(1, D_OUT_PAD)

    ones16 = jnp.ones((CH, 16), jnp.float32)
    zeros16 = jnp.zeros((ROWS_PER_TILE, 16), jnp.float32)
    zeros64 = jnp.zeros((ROWS_PER_TILE, D_HID), jnp.float32)
    zeros48 = jnp.zeros((ROWS_PER_TILE, D_OUT_PAD), jnp.float32)

    degp = _deg_call(dst3, ones16, zeros16)
    h1s, dis16 = _mm1(x_pad, W1, degp)
    aggp1 = _agg64_call(h1s, src3, dst3, zeros64)
    h2s = _mm2(aggp1, h1s, dis16, b1r, W2p)
    aggp2 = _agg48_call(h2s, src3, dst3, zeros48)
    outp = _post(aggp2, h2s, dis16, b2r)
    return outp[:N_NODES, :D_OUT]


def kernel(x, edge_index, W1, b1, W2, b2):
    return _run(x, edge_index, W1, b1, W2, b2)


# speedup vs baseline: 16.4230x; 16.4230x over previous
import functools

import jax
import jax.numpy as jnp
from jax import lax
from jax.experimental import pallas as pl
from jax.experimental.pallas import tpu as pltpu
from jax.experimental.pallas import tpu_sc as plsc

N_NODES = 10000
N_EDGES = 320000
D_IN = 128
D_HID = 64
D_OUT = 40
D_OUT_PAD = 48

NC = 2
NS = 16
NW = NC * NS

R = 10240
ROWS_PER_TILE = R // NS
PAD_ROW = N_NODES
CH = 128
TCH = 80
EPAD = NW * TCH * CH

BLK = 256
GRID = R // BLK

_MESH = plsc.VectorSubcoreMesh(
    core_axis_name="c", subcore_axis_name="s", num_cores=NC, num_subcores=NS
)
_SC_PARAMS = pltpu.CompilerParams(use_tc_tiling_on_sc=False)


def _zero_acc(zeros_hbm, acc, s):
    pltpu.sync_copy(zeros_hbm, acc.at[pl.ds(s * ROWS_PER_TILE, ROWS_PER_TILE)])


def _writeback(acc, out_hbm, c, s):
    base = s * ROWS_PER_TILE
    pltpu.sync_copy(
        acc.at[pl.ds(base, ROWS_PER_TILE)],
        out_hbm.at[c, pl.ds(base, ROWS_PER_TILE)],
    )


def _make_deg_kernel():
    @functools.partial(
        pl.kernel,
        mesh=_MESH,
        compiler_params=_SC_PARAMS,
        out_type=jax.ShapeDtypeStruct((NC, R, 16), jnp.float32),
        scratch_types=[
            pltpu.VMEM((TCH, CH), jnp.int32),
            pltpu.VMEM((CH, 16), jnp.float32),
            pltpu.VMEM_SHARED((R, 16), jnp.float32),
        ],
    )
    def deg_kernel(dst_hbm, ones_hbm, zeros_hbm, out_hbm, dst_v, ones_v, acc):
        c = lax.axis_index("c")
        s = lax.axis_index("s")
        w = c * NS + s
        _zero_acc(zeros_hbm, acc, s)
        pltpu.sync_copy(ones_hbm, ones_v)
        pltpu.sync_copy(dst_hbm.at[w], dst_v)
        plsc.subcore_barrier()

        def body(j, carry):
            pltpu.sync_copy(ones_v, acc.at[dst_v.at[j]], add=True)
            return carry

        lax.fori_loop(0, TCH, body, 0)
        plsc.subcore_barrier()
        _writeback(acc, out_hbm, c, s)

    return deg_kernel


def _make_agg_kernel(d):
    @functools.partial(
        pl.kernel,
        mesh=_MESH,
        compiler_params=_SC_PARAMS,
        out_type=jax.ShapeDtypeStruct((NC, R, d), jnp.float32),
        scratch_types=[
            pltpu.VMEM((TCH, CH), jnp.int32),
            pltpu.VMEM((TCH, CH), jnp.int32),
            pltpu.VMEM((CH, d), jnp.float32),
            pltpu.VMEM((CH, d), jnp.float32),
            pltpu.VMEM_SHARED((R, d), jnp.float32),
            pltpu.SemaphoreType.DMA,
            pltpu.SemaphoreType.DMA,
        ],
    )
    def agg_kernel(
        table_hbm, src_hbm, dst_hbm, zeros_hbm, out_hbm,
        src_v, dst_v, rows0, rows1, acc, sem0, sem1,
    ):
        c = lax.axis_index("c")
        s = lax.axis_index("s")
        w = c * NS + s
        _zero_acc(zeros_hbm, acc, s)
        pltpu.sync_copy(src_hbm.at[w], src_v)
        pltpu.sync_copy(dst_hbm.at[w], dst_v)
        plsc.subcore_barrier()

        bufs = (rows0, rows1)
        sems = (sem0, sem1)
        pltpu.async_copy(table_hbm.at[src_v.at[0]], rows0, sem0)
        pltpu.async_copy(table_hbm.at[src_v.at[1]], rows1, sem1)

        def body(j2, carry):
            for t in range(2):
                j = 2 * j2 + t
                pltpu.make_async_copy(
                    table_hbm.at[src_v.at[j]], bufs[t], sems[t]
                ).wait()
                pltpu.sync_copy(bufs[t], acc.at[dst_v.at[j]], add=True)

                @pl.when(j2 < TCH // 2 - 1)
                def _():
                    pltpu.async_copy(
                        table_hbm.at[src_v.at[j + 2]], bufs[t], sems[t]
                    )

            return carry

        lax.fori_loop(0, TCH // 2, body, 0)
        plsc.subcore_barrier()
        _writeback(acc, out_hbm, c, s)

    return agg_kernel


_deg_call = _make_deg_kernel()
_agg64_call = _make_agg_kernel(D_HID)
_agg48_call = _make_agg_kernel(D_OUT_PAD)


def _mm1_body(xb, w1b, degb, h1s_out, dis_out):
    deg = degb[0, :, 0:1] + degb[1, :, 0:1] + 1.0
    dis = lax.rsqrt(deg)
    h = jnp.dot(xb[...], w1b[...], preferred_element_type=jnp.float32)
    h1s_out[...] = h * dis
    dis_out[...] = jnp.broadcast_to(dis, (BLK, 16))


def _mm2_body(aggb, h1sb, disb, b1b, w2b, h2s_out):
    i = pl.program_id(0)
    dis = disb[:, 0:1]
    agg = aggb[0] + aggb[1] + h1sb[...]
    z = jnp.maximum(agg * dis + b1b[...], 0.0)
    h2 = jnp.dot(z, w2b[...], preferred_element_type=jnp.float32) * dis
    rows = lax.broadcasted_iota(jnp.int32, (BLK, D_OUT_PAD), 0) + i * BLK
    h2s_out[...] = jnp.where(rows < N_NODES, h2, 0.0)


def _post_body(aggb, h2sb, disb, b2b, out):
    agg = aggb[0] + aggb[1] + h2sb[...]
    out[...] = agg * disb[:, 0:1] + b2b[...]


def _mm1(x_pad, W1, degp):
    return pl.pallas_call(
        _mm1_body,
        grid=(GRID,),
        in_specs=[
            pl.BlockSpec((BLK, D_IN), lambda i: (i, 0)),
            pl.BlockSpec((D_IN, D_HID), lambda i: (0, 0)),
            pl.BlockSpec((NC, BLK, 16), lambda i: (0, i, 0)),
        ],
        out_specs=[
            pl.BlockSpec((BLK, D_HID), lambda i: (i, 0)),
            pl.BlockSpec((BLK, 16), lambda i: (i, 0)),
        ],
        out_shape=[
            jax.ShapeDtypeStruct((R, D_HID), jnp.float32),
            jax.ShapeDtypeStruct((R, 16), jnp.float32),
        ],
    )(x_pad, W1, degp)


def _mm2(aggp, h1s, dis16, b1r, W2p):
    return pl.pallas_call(
        _mm2_body,
        grid=(GRID,),
        in_specs=[
            pl.BlockSpec((NC, BLK, D_HID), lambda i: (0, i, 0)),
            pl.BlockSpec((BLK, D_HID), lambda i: (i, 0)),
            pl.BlockSpec((BLK, 16), lambda i: (i, 0)),
            pl.BlockSpec((1, D_HID), lambda i: (0, 0)),
            pl.BlockSpec((D_HID, D_OUT_PAD), lambda i: (0, 0)),
        ],
        out_specs=pl.BlockSpec((BLK, D_OUT_PAD), lambda i: (i, 0)),
        out_shape=jax.ShapeDtypeStruct((R, D_OUT_PAD), jnp.float32),
    )(aggp, h1s, dis16, b1r, W2p)


def _post(aggp, h2s, dis16, b2r):
    return pl.pallas_call(
        _post_body,
        grid=(GRID,),
        in_specs=[
            pl.BlockSpec((NC, BLK, D_OUT_PAD), lambda i: (0, i, 0)),
            pl.BlockSpec((BLK, D_OUT_PAD), lambda i: (i, 0)),
            pl.BlockSpec((BLK, 16), lambda i: (i, 0)),
            pl.BlockSpec((1, D_OUT_PAD), lambda i: (0, 0)),
        ],
        out_specs=pl.BlockSpec((BLK, D_OUT_PAD), lambda i: (i, 0)),
        out_shape=jax.ShapeDtypeStruct((R, D_OUT_PAD), jnp.float32),
    )(aggp, h2s, dis16, b2r)


@jax.jit
def _run(x, edge_index, W1, b1, W2, b2):
    edge_index = edge_index.astype(jnp.int32)
    pad = jnp.full((EPAD - N_EDGES,), PAD_ROW, dtype=jnp.int32)
    src3 = jnp.concatenate([edge_index[0], pad]).reshape(NW, TCH, CH)
    dst3 = jnp.concatenate([edge_index[1], pad]).reshape(NW, TCH, CH)

    x_pad = jnp.pad(x, ((0, R - N_NODES), (0, 0)))
    W2p = jnp.pad(W2, ((0, 0), (0, D_OUT_PAD - D_OUT)))
    b1r = b1.reshape(1, D_HID)
    b2r = jnp.pad(b2, (0, D_OUT_PAD - D_OUT)).reshape(1, D_OUT_PAD)

    ones16 = jnp.ones((CH, 16), jnp.float32)
    zeros16 = jnp.zeros((ROWS_PER_TILE, 16), jnp.float32)
    zeros64 = jnp.zeros((ROWS_PER_TILE, D_HID), jnp.float32)
    zeros48 = jnp.zeros((ROWS_PER_TILE, D_OUT_PAD), jnp.float32)

    degp = _deg_call(dst3, ones16, zeros16)
    h1s, dis16 = _mm1(x_pad, W1, degp)
    aggp1 = _agg64_call(h1s, src3, dst3, zeros64)
    h2s = _mm2(aggp1, h1s, dis16, b1r, W2p)
    aggp2 = _agg48_call(h2s, src3, dst3, zeros48)
    outp = _post(aggp2, h2s, dis16, b2r)
    return outp[:N_NODES, :D_OUT]


def kernel(x, edge_index, W1, b1, W2, b2):
    return _run(x, edge_index, W1, b1, W2, b2)


# speedup vs baseline: 16.4260x; 1.0002x over previous
import functools

import jax
import jax.numpy as jnp
from jax import lax
from jax.experimental import pallas as pl
from jax.experimental.pallas import tpu as pltpu
from jax.experimental.pallas import tpu_sc as plsc

N_NODES = 10000
N_EDGES = 320000
D_IN = 128
D_HID = 64
D_OUT = 40
D_OUT_PAD = 48

NC = 2
NS = 16
NW = NC * NS

R = 10240
ROWS_PER_TILE = R // NS
PAD_ROW = N_NODES
CH = 128
TCH = 80
NBUF = 8
KPF = 4
EPAD = NW * TCH * CH

BLK = 256
GRID = R // BLK

_MESH = plsc.VectorSubcoreMesh(
    core_axis_name="c", subcore_axis_name="s", num_cores=NC, num_subcores=NS
)
_SC_PARAMS = pltpu.CompilerParams(use_tc_tiling_on_sc=False)


def _zero_acc(zeros_hbm, acc, s):
    pltpu.sync_copy(zeros_hbm, acc.at[pl.ds(s * ROWS_PER_TILE, ROWS_PER_TILE)])


def _writeback(acc, out_hbm, c, s):
    base = s * ROWS_PER_TILE
    pltpu.sync_copy(
        acc.at[pl.ds(base, ROWS_PER_TILE)],
        out_hbm.at[c, pl.ds(base, ROWS_PER_TILE)],
    )


def _make_deg_kernel():
    @functools.partial(
        pl.kernel,
        mesh=_MESH,
        compiler_params=_SC_PARAMS,
        out_type=jax.ShapeDtypeStruct((NC, R, 16), jnp.float32),
        scratch_types=[
            pltpu.VMEM((TCH, CH), jnp.int32),
            pltpu.VMEM((CH, 16), jnp.float32),
            pltpu.VMEM_SHARED((R, 16), jnp.float32),
            pltpu.SemaphoreType.DMA,
        ],
    )
    def deg_kernel(dst_hbm, ones_hbm, zeros_hbm, out_hbm, dst_v, ones_v, acc, sem):
        c = lax.axis_index("c")
        s = lax.axis_index("s")
        w = c * NS + s
        _zero_acc(zeros_hbm, acc, s)
        pltpu.sync_copy(ones_hbm, ones_v)
        pltpu.sync_copy(dst_hbm.at[w], dst_v)
        plsc.subcore_barrier()

        def fire(j, carry):
            pltpu.async_copy(ones_v, acc.at[dst_v.at[j]], sem, add=True)
            return carry

        lax.fori_loop(0, TCH, fire, 0)

        def drain(j, carry):
            pltpu.make_async_copy(ones_v, acc.at[dst_v.at[j]], sem).wait()
            return carry

        lax.fori_loop(0, TCH, drain, 0)
        plsc.subcore_barrier()
        _writeback(acc, out_hbm, c, s)

    return deg_kernel


def _make_agg_kernel(d):
    @functools.partial(
        pl.kernel,
        mesh=_MESH,
        compiler_params=_SC_PARAMS,
        out_type=jax.ShapeDtypeStruct((NC, R, d), jnp.float32),
        scratch_types=[
            pltpu.VMEM((TCH, CH), jnp.int32),
            pltpu.VMEM((TCH, CH), jnp.int32),
            [pltpu.VMEM((CH, d), jnp.float32)] * NBUF,
            pltpu.VMEM_SHARED((R, d), jnp.float32),
            [pltpu.SemaphoreType.DMA] * NBUF,
            [pltpu.SemaphoreType.DMA] * NBUF,
        ],
    )
    def agg_kernel(
        table_hbm, src_hbm, dst_hbm, zeros_hbm, out_hbm,
        src_v, dst_v, bufs, acc, gsem, ssem,
    ):
        c = lax.axis_index("c")
        s = lax.axis_index("s")
        w = c * NS + s
        _zero_acc(zeros_hbm, acc, s)
        pltpu.sync_copy(src_hbm.at[w], src_v)
        pltpu.sync_copy(dst_hbm.at[w], dst_v)
        plsc.subcore_barrier()

        for j in range(KPF):
            pltpu.async_copy(table_hbm.at[src_v.at[j]], bufs[j], gsem[j])

        def body(jo, carry):
            for t in range(NBUF):
                j = NBUF * jo + t
                bp = (t + KPF) % NBUF
                pltpu.make_async_copy(
                    table_hbm.at[src_v.at[j]], bufs[t], gsem[t]
                ).wait()
                pltpu.async_copy(bufs[t], acc.at[dst_v.at[j]], ssem[t], add=True)

                @pl.when(j >= KPF)
                def _():
                    pltpu.make_async_copy(
                        bufs[bp], acc.at[dst_v.at[j - KPF]], ssem[bp]
                    ).wait()

                @pl.when(j + KPF < TCH)
                def _():
                    pltpu.async_copy(
                        table_hbm.at[src_v.at[j + KPF]], bufs[bp], gsem[bp]
                    )

            return carry

        lax.fori_loop(0, TCH // NBUF, body, 0)

        for j in range(TCH - KPF, TCH):
            b = j % NBUF
            pltpu.make_async_copy(bufs[b], acc.at[dst_v.at[j]], ssem[b]).wait()

        plsc.subcore_barrier()
        _writeback(acc, out_hbm, c, s)

    return agg_kernel


_deg_call = _make_deg_kernel()
_agg64_call = _make_agg_kernel(D_HID)
_agg48_call = _make_agg_kernel(D_OUT_PAD)


def _mm1_body(xb, w1b, degb, h1s_out, dis_out):
    deg = degb[0, :, 0:1] + degb[1, :, 0:1] + 1.0
    dis = lax.rsqrt(deg)
    h = jnp.dot(xb[...], w1b[...], preferred_element_type=jnp.float32)
    h1s_out[...] = h * dis
    dis_out[...] = jnp.broadcast_to(dis, (BLK, 16))


def _mm2_body(aggb, h1sb, disb, b1b, w2b, h2s_out):
    i = pl.program_id(0)
    dis = disb[:, 0:1]
    agg = aggb[0] + aggb[1] + h1sb[...]
    z = jnp.maximum(agg * dis + b1b[...], 0.0)
    h2 = jnp.dot(z, w2b[...], preferred_element_type=jnp.float32) * dis
    rows = lax.broadcasted_iota(jnp.int32, (BLK, D_OUT_PAD), 0) + i * BLK
    h2s_out[...] = jnp.where(rows < N_NODES, h2, 0.0)


def _post_body(aggb, h2sb, disb, b2b, out):
    agg = aggb[0] + aggb[1] + h2sb[...]
    out[...] = agg * disb[:, 0:1] + b2b[...]


def _mm1(x_pad, W1, degp):
    return pl.pallas_call(
        _mm1_body,
        grid=(GRID,),
        in_specs=[
            pl.BlockSpec((BLK, D_IN), lambda i: (i, 0)),
            pl.BlockSpec((D_IN, D_HID), lambda i: (0, 0)),
            pl.BlockSpec((NC, BLK, 16), lambda i: (0, i, 0)),
        ],
        out_specs=[
            pl.BlockSpec((BLK, D_HID), lambda i: (i, 0)),
            pl.BlockSpec((BLK, 16), lambda i: (i, 0)),
        ],
        out_shape=[
            jax.ShapeDtypeStruct((R, D_HID), jnp.float32),
            jax.ShapeDtypeStruct((R, 16), jnp.float32),
        ],
    )(x_pad, W1, degp)


def _mm2(aggp, h1s, dis16, b1r, W2p):
    return pl.pallas_call(
        _mm2_body,
        grid=(GRID,),
        in_specs=[
            pl.BlockSpec((NC, BLK, D_HID), lambda i: (0, i, 0)),
            pl.BlockSpec((BLK, D_HID), lambda i: (i, 0)),
            pl.BlockSpec((BLK, 16), lambda i: (i, 0)),
            pl.BlockSpec((1, D_HID), lambda i: (0, 0)),
            pl.BlockSpec((D_HID, D_OUT_PAD), lambda i: (0, 0)),
        ],
        out_specs=pl.BlockSpec((BLK, D_OUT_PAD), lambda i: (i, 0)),
        out_shape=jax.ShapeDtypeStruct((R, D_OUT_PAD), jnp.float32),
    )(aggp, h1s, dis16, b1r, W2p)


def _post(aggp, h2s, dis16, b2r):
    return pl.pallas_call(
        _post_body,
        grid=(GRID,),
        in_specs=[
            pl.BlockSpec((NC, BLK, D_OUT_PAD), lambda i: (0, i, 0)),
            pl.BlockSpec((BLK, D_OUT_PAD), lambda i: (i, 0)),
            pl.BlockSpec((BLK, 16), lambda i: (i, 0)),
            pl.BlockSpec((1, D_OUT_PAD), lambda i: (0, 0)),
        ],
        out_specs=pl.BlockSpec((BLK, D_OUT_PAD), lambda i: (i, 0)),
        out_shape=jax.ShapeDtypeStruct((R, D_OUT_PAD), jnp.float32),
    )(aggp, h2s, dis16, b2r)


@jax.jit
def _run(x, edge_index, W1, b1, W2, b2):
    edge_index = edge_index.astype(jnp.int32)
    pad = jnp.full((EPAD - N_EDGES,), PAD_ROW, dtype=jnp.int32)
    src3 = jnp.concatenate([edge_index[0], pad]).reshape(NW, TCH, CH)
    dst3 = jnp.concatenate([edge_index[1], pad]).reshape(NW, TCH, CH)

    x_pad = jnp.pad(x, ((0, R - N_NODES), (0, 0)))
    W2p = jnp.pad(W2, ((0, 0), (0, D_OUT_PAD - D_OUT)))
    b1r = b1.reshape(1, D_HID)
    b2r = jnp.pad(b2, (0, D_OUT_PAD - D_OUT)).reshape(1, D_OUT_PAD)

    ones16 = jnp.ones((CH, 16), jnp.float32)
    zeros16 = jnp.zeros((ROWS_PER_TILE, 16), jnp.float32)
    zeros64 = jnp.zeros((ROWS_PER_TILE, D_HID), jnp.float32)
    zeros48 = jnp.zeros((ROWS_PER_TILE, D_OUT_PAD), jnp.float32)

    degp = _deg_call(dst3, ones16, zeros16)
    h1s, dis16 = _mm1(x_pad, W1, degp)
    aggp1 = _agg64_call(h1s, src3, dst3, zeros64)
    h2s = _mm2(aggp1, h1s, dis16, b1r, W2p)
    aggp2 = _agg48_call(h2s, src3, dst3, zeros48)
    outp = _post(aggp2, h2s, dis16, b2r)
    return outp[:N_NODES, :D_OUT]


def kernel(x, edge_index, W1, b1, W2, b2):
    return _run(x, edge_index, W1, b1, W2, b2)


# speedup vs baseline: 43.3575x; 2.6396x over previous
import functools

import jax
import jax.numpy as jnp
from jax import lax
from jax.experimental import pallas as pl
from jax.experimental.pallas import tpu as pltpu
from jax.experimental.pallas import tpu_sc as plsc

N_NODES = 10000
N_EDGES = 320000
D_IN = 128
D_HID = 64
D_OUT = 40
D_OUT_PAD = 48

NC = 2
NS = 16
NW = NC * NS

R = 10240
ROWS_PER_TILE = R // NS
PAD_ROW = N_NODES
CH = 125
TCH = 80
NBUF = 8
KPF = 4

BLK = 1024
GRID = R // BLK

_MESH = plsc.VectorSubcoreMesh(
    core_axis_name="c", subcore_axis_name="s", num_cores=NC, num_subcores=NS
)
_SC_PARAMS = pltpu.CompilerParams(use_tc_tiling_on_sc=False)


def _zero_acc(zeros_hbm, acc, s):
    pltpu.sync_copy(zeros_hbm, acc.at[pl.ds(s * ROWS_PER_TILE, ROWS_PER_TILE)])


def _writeback(acc, out_hbm, c, s):
    base = s * ROWS_PER_TILE
    pltpu.sync_copy(
        acc.at[pl.ds(base, ROWS_PER_TILE)],
        out_hbm.at[c, pl.ds(base, ROWS_PER_TILE)],
    )


def _make_deg_kernel():
    @functools.partial(
        pl.kernel,
        mesh=_MESH,
        compiler_params=_SC_PARAMS,
        out_type=jax.ShapeDtypeStruct((NC, R, 16), jnp.float32),
        scratch_types=[
            pltpu.VMEM((TCH, CH), jnp.int32),
            pltpu.VMEM((CH, 16), jnp.float32),
            pltpu.VMEM_SHARED((R, 16), jnp.float32),
            pltpu.SemaphoreType.DMA,
        ],
    )
    def deg_kernel(edge_hbm, ones_hbm, zeros_hbm, out_hbm, dst_v, ones_v, acc, sem):
        c = lax.axis_index("c")
        s = lax.axis_index("s")
        w = c * NS + s
        _zero_acc(zeros_hbm, acc, s)
        pltpu.sync_copy(ones_hbm, ones_v)
        pltpu.sync_copy(edge_hbm.at[1, w], dst_v)
        plsc.subcore_barrier()

        def fire(j, carry):
            pltpu.async_copy(ones_v, acc.at[dst_v.at[j]], sem, add=True)
            return carry

        lax.fori_loop(0, TCH, fire, 0)

        def drain(j, carry):
            pltpu.make_async_copy(ones_v, acc.at[dst_v.at[j]], sem).wait()
            return carry

        lax.fori_loop(0, TCH, drain, 0)
        plsc.subcore_barrier()
        _writeback(acc, out_hbm, c, s)

    return deg_kernel


def _make_agg_kernel(d):
    @functools.partial(
        pl.kernel,
        mesh=_MESH,
        compiler_params=_SC_PARAMS,
        out_type=jax.ShapeDtypeStruct((NC, R, d), jnp.float32),
        scratch_types=[
            pltpu.VMEM((TCH, CH), jnp.int32),
            pltpu.VMEM((TCH, CH), jnp.int32),
            [pltpu.VMEM((CH, d), jnp.float32)] * NBUF,
            pltpu.VMEM_SHARED((R, d), jnp.float32),
            [pltpu.SemaphoreType.DMA] * NBUF,
            [pltpu.SemaphoreType.DMA] * NBUF,
        ],
    )
    def agg_kernel(
        table_hbm, edge_hbm, zeros_hbm, out_hbm,
        src_v, dst_v, bufs, acc, gsem, ssem,
    ):
        c = lax.axis_index("c")
        s = lax.axis_index("s")
        w = c * NS + s
        _zero_acc(zeros_hbm, acc, s)
        pltpu.sync_copy(edge_hbm.at[0, w], src_v)
        pltpu.sync_copy(edge_hbm.at[1, w], dst_v)
        plsc.subcore_barrier()

        for j in range(KPF):
            pltpu.async_copy(table_hbm.at[src_v.at[j]], bufs[j], gsem[j])

        def body(jo, carry):
            for t in range(NBUF):
                j = NBUF * jo + t
                bp = (t + KPF) % NBUF
                pltpu.make_async_copy(
                    table_hbm.at[src_v.at[j]], bufs[t], gsem[t]
                ).wait()
                pltpu.async_copy(bufs[t], acc.at[dst_v.at[j]], ssem[t], add=True)

                @pl.when(j >= KPF)
                def _():
                    pltpu.make_async_copy(
                        bufs[bp], acc.at[dst_v.at[j - KPF]], ssem[bp]
                    ).wait()

                @pl.when(j + KPF < TCH)
                def _():
                    pltpu.async_copy(
                        table_hbm.at[src_v.at[j + KPF]], bufs[bp], gsem[bp]
                    )

            return carry

        lax.fori_loop(0, TCH // NBUF, body, 0)

        for j in range(TCH - KPF, TCH):
            b = j % NBUF
            pltpu.make_async_copy(bufs[b], acc.at[dst_v.at[j]], ssem[b]).wait()

        plsc.subcore_barrier()
        _writeback(acc, out_hbm, c, s)

    return agg_kernel


_deg_call = _make_deg_kernel()
_agg64_call = _make_agg_kernel(D_HID)
_agg48_call = _make_agg_kernel(D_OUT_PAD)


def _mm1_body(xb, w1b, degb, h1s_out, dis_out):
    i = pl.program_id(0)
    deg = degb[0, :, 0:1] + degb[1, :, 0:1] + 1.0
    dis = lax.rsqrt(deg)
    h = jnp.dot(xb[...], w1b[...], preferred_element_type=jnp.float32)
    rows = lax.broadcasted_iota(jnp.int32, (BLK, D_HID), 0) + i * BLK
    h1s_out[...] = jnp.where(rows < N_NODES, h * dis, 0.0)
    dis_out[...] = jnp.broadcast_to(dis, (BLK, 16))


def _mm2_body(aggb, h1sb, disb, b1b, w2b, h2s_out):
    i = pl.program_id(0)
    dis = disb[:, 0:1]
    agg = aggb[0] + aggb[1] + h1sb[...]
    z = jnp.maximum(agg * dis + b1b[...], 0.0)
    h2 = jnp.dot(z, w2b[...], preferred_element_type=jnp.float32) * dis
    rows = lax.broadcasted_iota(jnp.int32, (BLK, D_OUT_PAD), 0) + i * BLK
    h2s_out[...] = jnp.where(rows < N_NODES, h2, 0.0)


def _post_body(aggb, h2sb, disb, b2b, out):
    agg = aggb[0] + aggb[1] + h2sb[...]
    out[...] = agg * disb[:, 0:1] + b2b[...]


def _mm1(x_pad, W1, degp):
    return pl.pallas_call(
        _mm1_body,
        grid=(GRID,),
        in_specs=[
            pl.BlockSpec((BLK, D_IN), lambda i: (i, 0)),
            pl.BlockSpec((D_IN, D_HID), lambda i: (0, 0)),
            pl.BlockSpec((NC, BLK, 16), lambda i: (0, i, 0)),
        ],
        out_specs=[
            pl.BlockSpec((BLK, D_HID), lambda i: (i, 0)),
            pl.BlockSpec((BLK, 16), lambda i: (i, 0)),
        ],
        out_shape=[
            jax.ShapeDtypeStruct((R, D_HID), jnp.float32),
            jax.ShapeDtypeStruct((R, 16), jnp.float32),
        ],
    )(x_pad, W1, degp)


def _mm2(aggp, h1s, dis16, b1r, W2p):
    return pl.pallas_call(
        _mm2_body,
        grid=(GRID,),
        in_specs=[
            pl.BlockSpec((NC, BLK, D_HID), lambda i: (0, i, 0)),
            pl.BlockSpec((BLK, D_HID), lambda i: (i, 0)),
            pl.BlockSpec((BLK, 16), lambda i: (i, 0)),
            pl.BlockSpec((1, D_HID), lambda i: (0, 0)),
            pl.BlockSpec((D_HID, D_OUT_PAD), lambda i: (0, 0)),
        ],
        out_specs=pl.BlockSpec((BLK, D_OUT_PAD), lambda i: (i, 0)),
        out_shape=jax.ShapeDtypeStruct((R, D_OUT_PAD), jnp.float32),
    )(aggp, h1s, dis16, b1r, W2p)


def _post(aggp, h2s, dis16, b2r):
    return pl.pallas_call(
        _post_body,
        grid=(GRID,),
        in_specs=[
            pl.BlockSpec((NC, BLK, D_OUT_PAD), lambda i: (0, i, 0)),
            pl.BlockSpec((BLK, D_OUT_PAD), lambda i: (i, 0)),
            pl.BlockSpec((BLK, 16), lambda i: (i, 0)),
            pl.BlockSpec((1, D_OUT_PAD), lambda i: (0, 0)),
        ],
        out_specs=pl.BlockSpec((BLK, D_OUT_PAD), lambda i: (i, 0)),
        out_shape=jax.ShapeDtypeStruct((R, D_OUT_PAD), jnp.float32),
    )(aggp, h2s, dis16, b2r)


@jax.jit
def _run(x, edge_index, W1, b1, W2, b2):
    edge4 = edge_index.astype(jnp.int32).reshape(2, NW, TCH, CH)

    W2p = jnp.pad(W2, ((0, 0), (0, D_OUT_PAD - D_OUT)))
    b1r = b1.reshape(1, D_HID)
    b2r = jnp.pad(b2, (0, D_OUT_PAD - D_OUT)).reshape(1, D_OUT_PAD)

    ones16 = jnp.ones((CH, 16), jnp.float32)
    zeros16 = jnp.zeros((ROWS_PER_TILE, 16), jnp.float32)
    zeros64 = jnp.zeros((ROWS_PER_TILE, D_HID), jnp.float32)
    zeros48 = jnp.zeros((ROWS_PER_TILE, D_OUT_PAD), jnp.float32)

    degp = _deg_call(edge4, ones16, zeros16)
    h1s, dis16 = _mm1(x, W1, degp)
    aggp1 = _agg64_call(h1s, edge4, zeros64)
    h2s = _mm2(aggp1, h1s, dis16, b1r, W2p)
    aggp2 = _agg48_call(h2s, edge4, zeros48)
    outp = _post(aggp2, h2s, dis16, b2r)
    return outp[:N_NODES, :D_OUT]


def kernel(x, edge_index, W1, b1, W2, b2):
    return _run(x, edge_index, W1, b1, W2, b2)


# speedup vs baseline: 45.4735x; 1.0488x over previous
import functools

import jax
import jax.numpy as jnp
from jax import lax
from jax.experimental import pallas as pl
from jax.experimental.pallas import tpu as pltpu
from jax.experimental.pallas import tpu_sc as plsc

N_NODES = 10000
N_EDGES = 320000
D_IN = 128
D_HID = 64
D_OUT = 40
D_OUT_PAD = 40

NC = 2
NS = 16
NW = NC * NS

R = 10240
ROWS_PER_TILE = R // NS
PAD_ROW = N_NODES
CH = 125
TCH = 80
NBUF = 8
KPF = 4

BLK = 2048
GRID = R // BLK

_MESH = plsc.VectorSubcoreMesh(
    core_axis_name="c", subcore_axis_name="s", num_cores=NC, num_subcores=NS
)
_SC_PARAMS = pltpu.CompilerParams(use_tc_tiling_on_sc=False)


def _zero_acc(zeros_hbm, acc, s):
    pltpu.sync_copy(zeros_hbm, acc.at[pl.ds(s * ROWS_PER_TILE, ROWS_PER_TILE)])


def _writeback(acc, out_hbm, c, s):
    base = s * ROWS_PER_TILE
    pltpu.sync_copy(
        acc.at[pl.ds(base, ROWS_PER_TILE)],
        out_hbm.at[c, pl.ds(base, ROWS_PER_TILE)],
    )


def _make_deg_kernel():
    @functools.partial(
        pl.kernel,
        mesh=_MESH,
        compiler_params=_SC_PARAMS,
        out_type=jax.ShapeDtypeStruct((NC, R, 16), jnp.float32),
        scratch_types=[
            pltpu.VMEM((TCH, CH), jnp.int32),
            pltpu.VMEM((CH, 16), jnp.float32),
            pltpu.VMEM_SHARED((R, 16), jnp.float32),
            pltpu.SemaphoreType.DMA,
        ],
    )
    def deg_kernel(edge_hbm, ones_hbm, zeros_hbm, out_hbm, dst_v, ones_v, acc, sem):
        c = lax.axis_index("c")
        s = lax.axis_index("s")
        w = c * NS + s
        _zero_acc(zeros_hbm, acc, s)
        pltpu.sync_copy(ones_hbm, ones_v)
        pltpu.sync_copy(edge_hbm.at[1, w], dst_v)
        plsc.subcore_barrier()

        def fire(j, carry):
            pltpu.async_copy(ones_v, acc.at[dst_v.at[j]], sem, add=True)
            return carry

        lax.fori_loop(0, TCH, fire, 0)

        def drain(j, carry):
            pltpu.make_async_copy(ones_v, acc.at[dst_v.at[j]], sem).wait()
            return carry

        lax.fori_loop(0, TCH, drain, 0)
        plsc.subcore_barrier()
        _writeback(acc, out_hbm, c, s)

    return deg_kernel


def _make_agg_kernel(d):
    @functools.partial(
        pl.kernel,
        mesh=_MESH,
        compiler_params=_SC_PARAMS,
        out_type=jax.ShapeDtypeStruct((NC, R, d), jnp.float32),
        scratch_types=[
            pltpu.VMEM((TCH, CH), jnp.int32),
            pltpu.VMEM((TCH, CH), jnp.int32),
            [pltpu.VMEM((CH, d), jnp.float32)] * NBUF,
            pltpu.VMEM_SHARED((R, d), jnp.float32),
            [pltpu.SemaphoreType.DMA] * NBUF,
            [pltpu.SemaphoreType.DMA] * NBUF,
        ],
    )
    def agg_kernel(
        table_hbm, edge_hbm, zeros_hbm, out_hbm,
        src_v, dst_v, bufs, acc, gsem, ssem,
    ):
        c = lax.axis_index("c")
        s = lax.axis_index("s")
        w = c * NS + s
        _zero_acc(zeros_hbm, acc, s)
        pltpu.sync_copy(edge_hbm.at[0, w], src_v)
        pltpu.sync_copy(edge_hbm.at[1, w], dst_v)
        plsc.subcore_barrier()

        for j in range(KPF):
            pltpu.async_copy(table_hbm.at[src_v.at[j]], bufs[j], gsem[j])

        def body(jo, carry):
            for t in range(NBUF):
                j = NBUF * jo + t
                bp = (t + KPF) % NBUF
                pltpu.make_async_copy(
                    table_hbm.at[src_v.at[j]], bufs[t], gsem[t]
                ).wait()
                pltpu.async_copy(bufs[t], acc.at[dst_v.at[j]], ssem[t], add=True)

                @pl.when(j >= KPF)
                def _():
                    pltpu.make_async_copy(
                        bufs[bp], acc.at[dst_v.at[j - KPF]], ssem[bp]
                    ).wait()

                @pl.when(j + KPF < TCH)
                def _():
                    pltpu.async_copy(
                        table_hbm.at[src_v.at[j + KPF]], bufs[bp], gsem[bp]
                    )

            return carry

        lax.fori_loop(0, TCH // NBUF, body, 0)

        for j in range(TCH - KPF, TCH):
            b = j % NBUF
            pltpu.make_async_copy(bufs[b], acc.at[dst_v.at[j]], ssem[b]).wait()

        plsc.subcore_barrier()
        _writeback(acc, out_hbm, c, s)

    return agg_kernel


_deg_call = _make_deg_kernel()
_agg64_call = _make_agg_kernel(D_HID)
_agg48_call = _make_agg_kernel(D_OUT_PAD)


def _mm1_body(xb, w1b, degb, h1s_out, dis_out):
    i = pl.program_id(0)
    deg = degb[0, :, 0:1] + degb[1, :, 0:1] + 1.0
    dis = lax.rsqrt(deg)
    h = jnp.dot(xb[...], w1b[...], preferred_element_type=jnp.float32)
    rows = lax.broadcasted_iota(jnp.int32, (BLK, D_HID), 0) + i * BLK
    h1s_out[...] = jnp.where(rows < N_NODES, h * dis, 0.0)
    dis_out[...] = jnp.broadcast_to(dis, (BLK, 16))


def _mm2_body(aggb, h1sb, disb, b1b, w2b, h2s_out):
    i = pl.program_id(0)
    dis = disb[:, 0:1]
    agg = aggb[0] + aggb[1] + h1sb[...]
    z = jnp.maximum(agg * dis + b1b[...], 0.0)
    h2 = jnp.dot(z, w2b[...], preferred_element_type=jnp.float32) * dis
    rows = lax.broadcasted_iota(jnp.int32, (BLK, D_OUT_PAD), 0) + i * BLK
    h2s_out[...] = jnp.where(rows < N_NODES, h2, 0.0)


def _post_body(aggb, h2sb, disb, b2b, out):
    agg = aggb[0] + aggb[1] + h2sb[...]
    out[...] = agg * disb[:, 0:1] + b2b[...]


def _mm1(x_pad, W1, degp):
    return pl.pallas_call(
        _mm1_body,
        grid=(GRID,),
        in_specs=[
            pl.BlockSpec((BLK, D_IN), lambda i: (i, 0)),
            pl.BlockSpec((D_IN, D_HID), lambda i: (0, 0)),
            pl.BlockSpec((NC, BLK, 16), lambda i: (0, i, 0)),
        ],
        out_specs=[
            pl.BlockSpec((BLK, D_HID), lambda i: (i, 0)),
            pl.BlockSpec((BLK, 16), lambda i: (i, 0)),
        ],
        out_shape=[
            jax.ShapeDtypeStruct((R, D_HID), jnp.float32),
            jax.ShapeDtypeStruct((R, 16), jnp.float32),
        ],
    )(x_pad, W1, degp)


def _mm2(aggp, h1s, dis16, b1r, W2p):
    return pl.pallas_call(
        _mm2_body,
        grid=(GRID,),
        in_specs=[
            pl.BlockSpec((NC, BLK, D_HID), lambda i: (0, i, 0)),
            pl.BlockSpec((BLK, D_HID), lambda i: (i, 0)),
            pl.BlockSpec((BLK, 16), lambda i: (i, 0)),
            pl.BlockSpec((1, D_HID), lambda i: (0, 0)),
            pl.BlockSpec((D_HID, D_OUT_PAD), lambda i: (0, 0)),
        ],
        out_specs=pl.BlockSpec((BLK, D_OUT_PAD), lambda i: (i, 0)),
        out_shape=jax.ShapeDtypeStruct((R, D_OUT_PAD), jnp.float32),
    )(aggp, h1s, dis16, b1r, W2p)


def _post(aggp, h2s, dis16, b2r):
    return pl.pallas_call(
        _post_body,
        grid=(GRID,),
        in_specs=[
            pl.BlockSpec((NC, BLK, D_OUT_PAD), lambda i: (0, i, 0)),
            pl.BlockSpec((BLK, D_OUT_PAD), lambda i: (i, 0)),
            pl.BlockSpec((BLK, 16), lambda i: (i, 0)),
            pl.BlockSpec((1, D_OUT_PAD), lambda i: (0, 0)),
        ],
        out_specs=pl.BlockSpec((BLK, D_OUT_PAD), lambda i: (i, 0)),
        out_shape=jax.ShapeDtypeStruct((R, D_OUT_PAD), jnp.float32),
    )(aggp, h2s, dis16, b2r)


@jax.jit
def _run(x, edge_index, W1, b1, W2, b2):
    edge4 = edge_index.astype(jnp.int32).reshape(2, NW, TCH, CH)

    W2p = jnp.pad(W2, ((0, 0), (0, D_OUT_PAD - D_OUT)))
    b1r = b1.reshape(1, D_HID)
    b2r = jnp.pad(b2, (0, D_OUT_PAD - D_OUT)).reshape(1, D_OUT_PAD)

    ones16 = jnp.ones((CH, 16), jnp.float32)
    zeros16 = jnp.zeros((ROWS_PER_TILE, 16), jnp.float32)
    zeros64 = jnp.zeros((ROWS_PER_TILE, D_HID), jnp.float32)
    zeros48 = jnp.zeros((ROWS_PER_TILE, D_OUT_PAD), jnp.float32)

    degp = _deg_call(edge4, ones16, zeros16)
    h1s, dis16 = _mm1(x, W1, degp)
    aggp1 = _agg64_call(h1s, edge4, zeros64)
    h2s = _mm2(aggp1, h1s, dis16, b1r, W2p)
    aggp2 = _agg48_call(h2s, edge4, zeros48)
    outp = _post(aggp2, h2s, dis16, b2r)
    return outp[:N_NODES, :D_OUT]


def kernel(x, edge_index, W1, b1, W2, b2):
    return _run(x, edge_index, W1, b1, W2, b2)


# speedup vs baseline: 47.2643x; 1.0394x over previous
import functools

import jax
import jax.numpy as jnp
from jax import lax
from jax.experimental import pallas as pl
from jax.experimental.pallas import tpu as pltpu
from jax.experimental.pallas import tpu_sc as plsc

N_NODES = 10000
N_EDGES = 320000
D_IN = 128
D_HID = 64
D_OUT = 40
D_OUT_PAD = 40

NC = 2
NS = 16
NW = NC * NS

R = 10240
ROWS_PER_TILE = R // NS
PAD_ROW = N_NODES
CH = 125
TCH = 80
NBUF = 8
KPF = 4

BLK = 2048
GRID = R // BLK

_MESH = plsc.VectorSubcoreMesh(
    core_axis_name="c", subcore_axis_name="s", num_cores=NC, num_subcores=NS
)
_SC_PARAMS = pltpu.CompilerParams(use_tc_tiling_on_sc=False)


def _zero_acc(zeros_hbm, acc, s):
    pltpu.sync_copy(zeros_hbm, acc.at[pl.ds(s * ROWS_PER_TILE, ROWS_PER_TILE)])


def _writeback(acc, out_hbm, c, s):
    base = s * ROWS_PER_TILE
    pltpu.sync_copy(
        acc.at[pl.ds(base, ROWS_PER_TILE)],
        out_hbm.at[c, pl.ds(base, ROWS_PER_TILE)],
    )


def _make_deg_kernel():
    @functools.partial(
        pl.kernel,
        mesh=_MESH,
        compiler_params=_SC_PARAMS,
        out_type=jax.ShapeDtypeStruct((NC, R), jnp.float32),
        scratch_types=[
            pltpu.VMEM((TCH, CH), jnp.int32),
            pltpu.VMEM((CH, 16), jnp.float32),
            pltpu.VMEM((ROWS_PER_TILE, 16), jnp.float32),
            pltpu.VMEM((ROWS_PER_TILE,), jnp.float32),
            pltpu.VMEM_SHARED((R, 16), jnp.float32),
            pltpu.SemaphoreType.DMA,
        ],
    )
    def deg_kernel(edge_hbm, ones_hbm, zeros_hbm, out_hbm, dst_v, ones_v,
                   vbuf, cvbuf, acc, sem):
        c = lax.axis_index("c")
        s = lax.axis_index("s")
        w = c * NS + s
        _zero_acc(zeros_hbm, acc, s)
        pltpu.sync_copy(ones_hbm, ones_v)
        pltpu.sync_copy(edge_hbm.at[1, w], dst_v)
        plsc.subcore_barrier()

        def fire(j, carry):
            pltpu.async_copy(ones_v, acc.at[dst_v.at[j]], sem, add=True)
            return carry

        lax.fori_loop(0, TCH, fire, 0)

        def drain(j, carry):
            pltpu.make_async_copy(ones_v, acc.at[dst_v.at[j]], sem).wait()
            return carry

        lax.fori_loop(0, TCH, drain, 0)
        plsc.subcore_barrier()

        base = s * ROWS_PER_TILE
        pltpu.sync_copy(acc.at[pl.ds(base, ROWS_PER_TILE)], vbuf)
        lane = lax.iota(jnp.int32, 16)

        def compact(k, carry):
            vals = jnp.zeros((16,), jnp.float32)
            for jj in range(16):
                vals = jnp.where(lane == jj, vbuf[k * 16 + jj, :], vals)
            cvbuf[pl.ds(k * 16, 16)] = vals
            return carry

        lax.fori_loop(0, ROWS_PER_TILE // 16, compact, 0)
        pltpu.sync_copy(cvbuf, out_hbm.at[c, pl.ds(base, ROWS_PER_TILE)])

    return deg_kernel


def _make_agg_kernel(d):
    @functools.partial(
        pl.kernel,
        mesh=_MESH,
        compiler_params=_SC_PARAMS,
        out_type=jax.ShapeDtypeStruct((NC, R, d), jnp.float32),
        scratch_types=[
            pltpu.VMEM((TCH, CH), jnp.int32),
            pltpu.VMEM((TCH, CH), jnp.int32),
            [pltpu.VMEM((CH, d), jnp.float32)] * NBUF,
            pltpu.VMEM_SHARED((R, d), jnp.float32),
            [pltpu.SemaphoreType.DMA] * NBUF,
            [pltpu.SemaphoreType.DMA] * NBUF,
        ],
    )
    def agg_kernel(
        table_hbm, edge_hbm, zeros_hbm, out_hbm,
        src_v, dst_v, bufs, acc, gsem, ssem,
    ):
        c = lax.axis_index("c")
        s = lax.axis_index("s")
        w = c * NS + s
        _zero_acc(zeros_hbm, acc, s)
        pltpu.sync_copy(edge_hbm.at[0, w], src_v)
        pltpu.sync_copy(edge_hbm.at[1, w], dst_v)
        plsc.subcore_barrier()

        for j in range(KPF):
            pltpu.async_copy(table_hbm.at[src_v.at[j]], bufs[j], gsem[j])

        def body(jo, carry):
            for t in range(NBUF):
                j = NBUF * jo + t
                bp = (t + KPF) % NBUF
                pltpu.make_async_copy(
                    table_hbm.at[src_v.at[j]], bufs[t], gsem[t]
                ).wait()
                pltpu.async_copy(bufs[t], acc.at[dst_v.at[j]], ssem[t], add=True)

                @pl.when(j >= KPF)
                def _():
                    pltpu.make_async_copy(
                        bufs[bp], acc.at[dst_v.at[j - KPF]], ssem[bp]
                    ).wait()

                @pl.when(j + KPF < TCH)
                def _():
                    pltpu.async_copy(
                        table_hbm.at[src_v.at[j + KPF]], bufs[bp], gsem[bp]
                    )

            return carry

        lax.fori_loop(0, TCH // NBUF, body, 0)

        for j in range(TCH - KPF, TCH):
            b = j % NBUF
            pltpu.make_async_copy(bufs[b], acc.at[dst_v.at[j]], ssem[b]).wait()

        plsc.subcore_barrier()
        _writeback(acc, out_hbm, c, s)

    return agg_kernel


_deg_call = _make_deg_kernel()
_agg64_call = _make_agg_kernel(D_HID)
_agg48_call = _make_agg_kernel(D_OUT_PAD)


def _mm1_body(xb, w1b, degb, h1s_out, dis_out):
    i = pl.program_id(0)
    deg = degb[0:1, :] + degb[1:2, :] + 1.0
    dis = jnp.transpose(lax.rsqrt(deg))
    h = jnp.dot(xb[...], w1b[...], preferred_element_type=jnp.float32)
    rows = lax.broadcasted_iota(jnp.int32, (BLK, D_HID), 0) + i * BLK
    h1s_out[...] = jnp.where(rows < N_NODES, h * dis, 0.0)
    dis_out[...] = jnp.broadcast_to(dis, (BLK, 16))


def _mm2_body(aggb, h1sb, disb, b1b, w2b, h2s_out):
    i = pl.program_id(0)
    dis = disb[:, 0:1]
    agg = aggb[0] + aggb[1] + h1sb[...]
    z = jnp.maximum(agg * dis + b1b[...], 0.0)
    h2 = jnp.dot(z, w2b[...], preferred_element_type=jnp.float32) * dis
    rows = lax.broadcasted_iota(jnp.int32, (BLK, D_OUT_PAD), 0) + i * BLK
    h2s_out[...] = jnp.where(rows < N_NODES, h2, 0.0)


def _post_body(aggb, h2sb, disb, b2b, out):
    agg = aggb[0] + aggb[1] + h2sb[...]
    out[...] = agg * disb[:, 0:1] + b2b[...]


def _mm1(x_pad, W1, degp):
    return pl.pallas_call(
        _mm1_body,
        grid=(GRID,),
        in_specs=[
            pl.BlockSpec((BLK, D_IN), lambda i: (i, 0)),
            pl.BlockSpec((D_IN, D_HID), lambda i: (0, 0)),
            pl.BlockSpec((NC, BLK), lambda i: (0, i)),
        ],
        out_specs=[
            pl.BlockSpec((BLK, D_HID), lambda i: (i, 0)),
            pl.BlockSpec((BLK, 16), lambda i: (i, 0)),
        ],
        out_shape=[
            jax.ShapeDtypeStruct((R, D_HID), jnp.float32),
            jax.ShapeDtypeStruct((R, 16), jnp.float32),
        ],
    )(x_pad, W1, degp)


def _mm2(aggp, h1s, dis16, b1r, W2p):
    return pl.pallas_call(
        _mm2_body,
        grid=(GRID,),
        in_specs=[
            pl.BlockSpec((NC, BLK, D_HID), lambda i: (0, i, 0)),
            pl.BlockSpec((BLK, D_HID), lambda i: (i, 0)),
            pl.BlockSpec((BLK, 16), lambda i: (i, 0)),
            pl.BlockSpec((1, D_HID), lambda i: (0, 0)),
            pl.BlockSpec((D_HID, D_OUT_PAD), lambda i: (0, 0)),
        ],
        out_specs=pl.BlockSpec((BLK, D_OUT_PAD), lambda i: (i, 0)),
        out_shape=jax.ShapeDtypeStruct((R, D_OUT_PAD), jnp.float32),
    )(aggp, h1s, dis16, b1r, W2p)


PBLK = 2000


def _post(aggp, h2s, dis16, b2r):
    return pl.pallas_call(
        _post_body,
        grid=(N_NODES // PBLK,),
        in_specs=[
            pl.BlockSpec((NC, PBLK, D_OUT_PAD), lambda i: (0, i, 0)),
            pl.BlockSpec((PBLK, D_OUT_PAD), lambda i: (i, 0)),
            pl.BlockSpec((PBLK, 16), lambda i: (i, 0)),
            pl.BlockSpec((1, D_OUT_PAD), lambda i: (0, 0)),
        ],
        out_specs=pl.BlockSpec((PBLK, D_OUT_PAD), lambda i: (i, 0)),
        out_shape=jax.ShapeDtypeStruct((N_NODES, D_OUT_PAD), jnp.float32),
    )(aggp, h2s, dis16, b2r)


@jax.jit
def _run(x, edge_index, W1, b1, W2, b2):
    edge4 = edge_index.astype(jnp.int32).reshape(2, NW, TCH, CH)

    W2p = jnp.pad(W2, ((0, 0), (0, D_OUT_PAD - D_OUT)))
    b1r = b1.reshape(1, D_HID)
    b2r = jnp.pad(b2, (0, D_OUT_PAD - D_OUT)).reshape(1, D_OUT_PAD)

    ones16 = jnp.ones((CH, 16), jnp.float32)
    zeros64 = jnp.zeros((ROWS_PER_TILE, D_HID), jnp.float32)

    degp = _deg_call(edge4, ones16, zeros64[:, :16])
    h1s, dis16 = _mm1(x, W1, degp)
    aggp1 = _agg64_call(h1s, edge4, zeros64)
    h2s = _mm2(aggp1, h1s, dis16, b1r, W2p)
    aggp2 = _agg48_call(h2s, edge4, zeros64[:, :D_OUT_PAD])
    return _post(aggp2, h2s, dis16, b2r)


def kernel(x, edge_index, W1, b1, W2, b2):
    return _run(x, edge_index, W1, b1, W2, b2)
